# Initial kernel scaffold; baseline (speedup 1.0000x reference)
import jax
import jax.numpy as jnp
from jax.experimental import pallas as pl


def kernel(x, fc1_W, fc1_b, fc2_W, fc2_b, g1_Ws, g1_bs, g1_Wh, g1_bh, g1_Wo1, g1_Wo2, g1_bo2, g2_Ws, g2_bs, g2_Wh, g2_bh, g2_Wo1, g2_Wo2, g2_bo2, g3_Ws, g3_bs, g3_Wh, g3_bh, g3_Wo1, g3_Wo2, g3_bo2, g4_Ws, g4_bs, g4_Wh, g4_bh, g4_Wo1, g4_Wo2, g4_bo2, fc3_W, fc3_b, fc4_W, fc4_b):
    raise NotImplementedError("write your pallas kernel here")


# speedup vs baseline: 3.0120x; 3.0120x over previous
import functools

import jax
import jax.numpy as jnp
from jax.experimental import pallas as pl
from jax.experimental.pallas import tpu as pltpu
from jax.experimental.pallas import tpu_sc as plsc

N = 10000
NPAD = 10240
BR = 256
K = 16
PROP = 64
BIGV = 3.0e38
PADV = 1.0e30


def _dot(a, b):
    return jax.lax.dot_general(
        a.astype(jnp.bfloat16), b.astype(jnp.bfloat16),
        (((1,), (0,)), ((), ())),
        preferred_element_type=jnp.float32)


def _dot_t(a, b):
    return jax.lax.dot_general(
        a.astype(jnp.bfloat16), b.astype(jnp.bfloat16),
        (((1,), (1,)), ((), ())),
        preferred_element_type=jnp.float32)


def _mlp_in(xp, W1, b1, W2, b2):
    def body(x_ref, w1_ref, b1_ref, w2_ref, b2_ref, o_ref):
        t = jax.nn.relu(_dot(x_ref[...], w1_ref[...]) + b1_ref[...])
        o_ref[...] = jax.nn.relu(_dot(t, w2_ref[...]) + b2_ref[...])
    return pl.pallas_call(
        body,
        out_shape=jax.ShapeDtypeStruct((xp.shape[0], W2.shape[1]), jnp.float32),
    )(xp, W1, b1, W2, b2)


def _mlp_out(xp, W3, b3, W4, b4):
    def body(x_ref, w3_ref, b3_ref, w4_ref, b4_ref, o_ref):
        t = jax.nn.relu(_dot(x_ref[...], w3_ref[...]) + b3_ref[...])
        o_ref[...] = _dot(t, w4_ref[...]) + b4_ref[...]
    return pl.pallas_call(
        body,
        out_shape=jax.ShapeDtypeStruct((xp.shape[0], W4.shape[1]), jnp.float32),
    )(xp, W3, b3, W4, b4)


def _prep(xp, Wsp, bs8, Wh, bh):
    NP = xp.shape[0]

    def body(x_ref, ws_ref, bs_ref, wh_ref, bh_ref, s_ref, h_ref):
        x = x_ref[...]
        s_ref[...] = _dot(x, ws_ref[...]) + bs_ref[...]
        h_ref[...] = _dot(x, wh_ref[...]) + bh_ref[...]

    return pl.pallas_call(
        body,
        out_shape=[
            jax.ShapeDtypeStruct((NP, 8), jnp.float32),
            jax.ShapeDtypeStruct((NP, Wh.shape[1]), jnp.float32),
        ],
    )(xp, Wsp, bs8, Wh, bh)


def _knn(s, sqr2, sqc8):
    NP = s.shape[0]
    nblk = NP // BR

    def body(sb_ref, sqr_ref, s_ref, sq_ref, idx_ref, v_ref):
        sb = sb_ref[...]
        dot = _dot_t(sb, s_ref[...])
        sqr = sqr_ref[...]
        sqc = sq_ref[0:1, :]
        d2 = sqr + sqc - 2.0 * dot
        col = jax.lax.broadcasted_iota(jnp.int32, (BR, NP), 1)
        lk = jax.lax.broadcasted_iota(jnp.int32, (BR, K), 1)

        def step(k, carry):
            D, V, I = carry
            m = jnp.min(D, axis=1, keepdims=True)
            cand = jnp.where(D == m, col, jnp.int32(2147483647))
            a = jnp.min(cand, axis=1, keepdims=True)
            D = jnp.where(col == a, BIGV, D)
            V = jnp.where(lk == k, m, V)
            I = jnp.where(lk == k, a, I)
            return D, V, I

        _, V, I = jax.lax.fori_loop(
            0, K, step,
            (d2, jnp.zeros((BR, K), jnp.float32), jnp.zeros((BR, K), jnp.int32)))
        idx_ref[...] = I
        v_ref[...] = V

    return pl.pallas_call(
        body,
        grid=(nblk,),
        in_specs=[
            pl.BlockSpec((BR, 8), lambda i: (i, 0)),
            pl.BlockSpec((BR, 1), lambda i: (i, 0)),
            pl.BlockSpec((NP, 8), lambda i: (0, 0)),
            pl.BlockSpec((8, NP), lambda i: (0, 0)),
        ],
        out_specs=[
            pl.BlockSpec((BR, K), lambda i: (i, 0)),
            pl.BlockSpec((BR, K), lambda i: (i, 0)),
        ],
        out_shape=[
            jax.ShapeDtypeStruct((NP, K), jnp.int32),
            jax.ShapeDtypeStruct((NP, K), jnp.float32),
        ],
    )(s, sqr2, s, sqc8)


def _agg(h, idx_flat, w_flat):
    NP, HW = h.shape
    rpw = NP // 32
    CB = 16
    nchunks = rpw // CB
    ipc = CB * K

    mesh = plsc.VectorSubcoreMesh(core_axis_name="c", subcore_axis_name="s")

    @functools.partial(
        pl.kernel,
        mesh=mesh,
        out_type=jax.ShapeDtypeStruct((NP, 2 * PROP), jnp.float32),
        scratch_types=[
            pltpu.VMEM((ipc,), jnp.int32),
            pltpu.VMEM((ipc,), jnp.float32),
            pltpu.VMEM((ipc, HW), jnp.float32),
            pltpu.VMEM((CB, 2 * PROP), jnp.float32),
            pltpu.SemaphoreType.DMA,
        ],
    )
    def k(h_hbm, idx_hbm, w_hbm, out_hbm, idxv, wv, gv, ov, sem):
        cid = jax.lax.axis_index("c")
        sid = jax.lax.axis_index("s")
        wid = sid * 2 + cid

        @pl.loop(0, nchunks)
        def _(c):
            base_row = wid * rpw + c * CB
            base_i = base_row * K
            pltpu.sync_copy(idx_hbm.at[pl.ds(base_i, ipc)], idxv)
            pltpu.sync_copy(w_hbm.at[pl.ds(base_i, ipc)], wv)
            pltpu.async_copy(h_hbm.at[idxv], gv, sem).wait()

            @pl.loop(0, CB)
            def _(r):
                acc = [jnp.zeros((16,), jnp.float32) for _ in range(4)]
                mx = [jnp.full((16,), -BIGV, jnp.float32) for _ in range(4)]
                wrow = wv[pl.ds(r * K, K)]
                for kk in range(K):
                    wk = wrow[kk]
                    for fb in range(4):
                        t = gv[r * K + kk, pl.ds(fb * 16, 16)] * wk
                        acc[fb] = acc[fb] + t
                        mx[fb] = jnp.maximum(mx[fb], t)
                for fb in range(4):
                    ov[r, pl.ds(fb * 16, 16)] = acc[fb] * (1.0 / K)
                    ov[r, pl.ds(PROP + fb * 16, 16)] = mx[fb]

            pltpu.sync_copy(ov, out_hbm.at[pl.ds(base_row, CB)])

    return k(h, idx_flat, w_flat)


def _gn_out(xp, agg, Wo1, Wo2, bo2):
    def body(x_ref, a_ref, w1_ref, w2_ref, b2_ref, o_ref):
        o_ref[...] = (_dot(x_ref[...], w1_ref[...])
                      + (_dot(a_ref[...], w2_ref[...]) + b2_ref[...]))
    return pl.pallas_call(
        body,
        out_shape=jax.ShapeDtypeStruct((xp.shape[0], Wo2.shape[1]), jnp.float32),
    )(xp, agg, Wo1, Wo2, bo2)


def kernel(x, fc1_W, fc1_b, fc2_W, fc2_b,
           g1_Ws, g1_bs, g1_Wh, g1_bh, g1_Wo1, g1_Wo2, g1_bo2,
           g2_Ws, g2_bs, g2_Wh, g2_bh, g2_Wo1, g2_Wo2, g2_bo2,
           g3_Ws, g3_bs, g3_Wh, g3_bh, g3_Wo1, g3_Wo2, g3_bo2,
           g4_Ws, g4_bs, g4_Wh, g4_bh, g4_Wo1, g4_Wo2, g4_bo2,
           fc3_W, fc3_b, fc4_W, fc4_b):
    n = x.shape[0]
    npad = -(-n // 512) * 512
    xp = jnp.pad(x, ((0, npad - n), (0, 0)))

    def r2(b):
        return b.reshape(1, -1)

    xc = _mlp_in(xp, fc1_W, r2(fc1_b), fc2_W, r2(fc2_b))

    layers = [
        (g1_Ws, g1_bs, g1_Wh, g1_bh, g1_Wo1, g1_Wo2, g1_bo2),
        (g2_Ws, g2_bs, g2_Wh, g2_bh, g2_Wo1, g2_Wo2, g2_bo2),
        (g3_Ws, g3_bs, g3_Wh, g3_bh, g3_Wo1, g3_Wo2, g3_bo2),
        (g4_Ws, g4_bs, g4_Wh, g4_bh, g4_Wo1, g4_Wo2, g4_bo2),
    ]
    for (Ws, bs, Wh, bh, Wo1, Wo2, bo2) in layers:
        Wsp = jnp.pad(Ws, ((0, 0), (0, 8 - Ws.shape[1])))
        bs8 = jnp.pad(bs, (0, 8 - bs.shape[0])).reshape(1, 8)
        Whp = jnp.pad(Wh, ((0, 0), (0, 128 - Wh.shape[1])))
        bhp = jnp.pad(bh, (0, 128 - bh.shape[0])).reshape(1, 128)
        s, h = _prep(xc, Wsp, bs8, Whp, bhp)
        s4 = s[:, :4]
        sq = jnp.sum(s4 * s4, axis=-1)
        npd = s.shape[0]
        sqr2 = sq.reshape(npd, 1)
        sqc = sq + jnp.where(jnp.arange(npd) >= n, PADV, 0.0)
        sqc8 = jnp.broadcast_to(sqc.reshape(1, npd), (8, npd))
        idx, v = _knn(s, sqr2, sqc8)
        w = jnp.exp(-10.0 * jnp.maximum(v, 0.0))
        agg = _agg(h, idx.reshape(-1), w.reshape(-1))
        xc = _gn_out(xc, agg, Wo1, Wo2, r2(bo2))

    y = _mlp_out(xc, fc3_W, r2(fc3_b), fc4_W, r2(fc4_b))
    return y[:n]
